# Initial kernel scaffold; baseline (speedup 1.0000x reference)
import jax
import jax.numpy as jnp
from jax.experimental import pallas as pl


def kernel(x, edge_index, W1, b1, W2, b2):
    raise NotImplementedError("write your pallas kernel here")


# speedup vs baseline: 25.6899x; 25.6899x over previous
import functools

import jax
import jax.numpy as jnp
from jax import lax
from jax.experimental import pallas as pl
from jax.experimental.pallas import tpu as pltpu
from jax.experimental.pallas import tpu_sc as plsc

N_NODES = 10000
N_EDGES = 320000
IN_DIM = 128
HID_DIM = 64
OUT_DIM = 64

NC = 2
NS = 16
NTILES = NC * NS
EPT = N_EDGES // NTILES
CH = 100
NCH = EPT // CH
RPT = 624
RPT_LAST = N_NODES - 15 * RPT

@functools.cache
def _mesh():
    return plsc.VectorSubcoreMesh(core_axis_name="c", subcore_axis_name="s",
                                  num_cores=NC, num_subcores=NS)


def _deg_body(col3, deg_out, cidx, ones_v, zbuf, dbuf, shared_deg):
    c = lax.axis_index("c")
    s = lax.axis_index("s")
    for i in range(7):
        ones_v[pl.ds(i * 16, 16)] = jnp.full((16,), 1.0, jnp.float32)
    for i in range(63):
        zbuf[pl.ds(i * 16, 16)] = jnp.zeros((16,), jnp.float32)

    @pl.when(s < 10)
    def _():
        pltpu.sync_copy(zbuf.at[pl.ds(0, 1000)],
                        shared_deg.at[pl.ds(s * 1000, 1000)])

    plsc.subcore_barrier()

    def outer(k, carry):
        pltpu.sync_copy(col3.at[2 * s + k], cidx)

        def inner(j, carry2):
            pltpu.sync_copy(ones_v.at[pl.ds(0, CH)],
                            shared_deg.at[cidx.at[j]], add=True)
            return carry2

        lax.fori_loop(0, NCH, inner, 0)
        return carry

    lax.fori_loop(0, 2, outer, 0)
    plsc.subcore_barrier()

    @pl.when(s < 5)
    def _():
        base = c * 5000 + s * 1000
        pltpu.sync_copy(shared_deg.at[pl.ds(base, 1000)], dbuf)
        pltpu.sync_copy(dbuf, deg_out.at[pl.ds(base, 1000)])


@functools.cache
def _deg_call():
    return pl.kernel(
        _deg_body,
        out_type=jax.ShapeDtypeStruct((N_NODES,), jnp.float32),
        mesh=_mesh(),
        scratch_types=[
            pltpu.VMEM((NCH, CH), jnp.int32),
            pltpu.VMEM((112,), jnp.float32),
            pltpu.VMEM((1008,), jnp.float32),
            pltpu.VMEM((1000,), jnp.float32),
            pltpu.VMEM_SHARED((N_NODES,), jnp.float32),
        ],
        compiler_params=pltpu.CompilerParams(use_tc_tiling_on_sc=False),
    )


def _prop_body(hp, row3, col3, zeros2, out, ridx, cidx, rowbuf, shared_acc):
    c = lax.axis_index("c")
    s = lax.axis_index("s")
    wid = c * NS + s
    r0 = s * RPT

    @pl.when(s < 15)
    def _():
        pltpu.sync_copy(zeros2.at[pl.ds(r0, RPT)],
                        shared_acc.at[pl.ds(r0, RPT)])

    @pl.when(s == 15)
    def _():
        pltpu.sync_copy(zeros2.at[pl.ds(15 * RPT, RPT_LAST)],
                        shared_acc.at[pl.ds(15 * RPT, RPT_LAST)])

    pltpu.sync_copy(row3.at[wid], ridx)
    pltpu.sync_copy(col3.at[wid], cidx)
    plsc.subcore_barrier()

    def body(j, carry):
        pltpu.sync_copy(hp.at[ridx.at[j]], rowbuf)
        pltpu.sync_copy(rowbuf, shared_acc.at[cidx.at[j]], add=True)
        return carry

    lax.fori_loop(0, NCH, body, 0)
    plsc.subcore_barrier()

    @pl.when(s < 15)
    def _():
        pltpu.sync_copy(shared_acc.at[pl.ds(r0, RPT)],
                        out.at[c, pl.ds(r0, RPT)])

    @pl.when(s == 15)
    def _():
        pltpu.sync_copy(shared_acc.at[pl.ds(15 * RPT, RPT_LAST)],
                        out.at[c, pl.ds(15 * RPT, RPT_LAST)])


@functools.cache
def _prop_call():
    return pl.kernel(
        _prop_body,
        out_type=jax.ShapeDtypeStruct((NC, N_NODES, HID_DIM), jnp.float32),
        mesh=_mesh(),
        scratch_types=[
            pltpu.VMEM((NCH, CH), jnp.int32),
            pltpu.VMEM((NCH, CH), jnp.int32),
            pltpu.VMEM((CH, HID_DIM), jnp.float32),
            pltpu.VMEM_SHARED((N_NODES, HID_DIM), jnp.float32),
        ],
        compiler_params=pltpu.CompilerParams(use_tc_tiling_on_sc=False),
    )


BLK = 2000
GRID = N_NODES // BLK


def _mm_body(x_ref, w_ref, o_ref):
    o_ref[...] = jnp.dot(x_ref[...], w_ref[...],
                         preferred_element_type=jnp.float32)


_mm = pl.pallas_call(
    _mm_body,
    grid=(GRID,),
    in_specs=[
        pl.BlockSpec((BLK, IN_DIM), lambda i: (i, 0)),
        pl.BlockSpec((IN_DIM, HID_DIM), lambda i: (0, 0)),
    ],
    out_specs=pl.BlockSpec((BLK, HID_DIM), lambda i: (i, 0)),
    out_shape=jax.ShapeDtypeStruct((N_NODES, HID_DIM), jnp.float32),
)


def _prescale_body(deg_ref, h_ref, dinv_ref, hp_ref):
    dinv = lax.rsqrt(deg_ref[...] + 1.0)
    dinv_ref[...] = dinv
    hp_ref[...] = h_ref[...] * dinv


_prescale = pl.pallas_call(
    _prescale_body,
    grid=(GRID,),
    in_specs=[
        pl.BlockSpec((BLK, 1), lambda i: (i, 0)),
        pl.BlockSpec((BLK, HID_DIM), lambda i: (i, 0)),
    ],
    out_specs=[
        pl.BlockSpec((BLK, 1), lambda i: (i, 0)),
        pl.BlockSpec((BLK, HID_DIM), lambda i: (i, 0)),
    ],
    out_shape=[
        jax.ShapeDtypeStruct((N_NODES, 1), jnp.float32),
        jax.ShapeDtypeStruct((N_NODES, HID_DIM), jnp.float32),
    ],
)


def _combine1_body(p_ref, h_ref, dinv_ref, b_ref, w_ref, h2_ref, hp2_ref):
    dinv = dinv_ref[...]
    psum = p_ref[0] + p_ref[1]
    out1 = jnp.maximum(dinv * psum + dinv * dinv * h_ref[...] + b_ref[...],
                       0.0)
    h2 = jnp.dot(out1, w_ref[...], preferred_element_type=jnp.float32)
    h2_ref[...] = h2
    hp2_ref[...] = h2 * dinv


_combine1 = pl.pallas_call(
    _combine1_body,
    grid=(GRID,),
    in_specs=[
        pl.BlockSpec((NC, BLK, HID_DIM), lambda i: (0, i, 0)),
        pl.BlockSpec((BLK, HID_DIM), lambda i: (i, 0)),
        pl.BlockSpec((BLK, 1), lambda i: (i, 0)),
        pl.BlockSpec((1, HID_DIM), lambda i: (0, 0)),
        pl.BlockSpec((HID_DIM, OUT_DIM), lambda i: (0, 0)),
    ],
    out_specs=[
        pl.BlockSpec((BLK, OUT_DIM), lambda i: (i, 0)),
        pl.BlockSpec((BLK, OUT_DIM), lambda i: (i, 0)),
    ],
    out_shape=[
        jax.ShapeDtypeStruct((N_NODES, OUT_DIM), jnp.float32),
        jax.ShapeDtypeStruct((N_NODES, OUT_DIM), jnp.float32),
    ],
)


def _combine2_body(p_ref, h_ref, dinv_ref, b_ref, o_ref):
    dinv = dinv_ref[...]
    psum = p_ref[0] + p_ref[1]
    o_ref[...] = dinv * psum + dinv * dinv * h_ref[...] + b_ref[...]


_combine2 = pl.pallas_call(
    _combine2_body,
    grid=(GRID,),
    in_specs=[
        pl.BlockSpec((NC, BLK, OUT_DIM), lambda i: (0, i, 0)),
        pl.BlockSpec((BLK, OUT_DIM), lambda i: (i, 0)),
        pl.BlockSpec((BLK, 1), lambda i: (i, 0)),
        pl.BlockSpec((1, OUT_DIM), lambda i: (0, 0)),
    ],
    out_specs=pl.BlockSpec((BLK, OUT_DIM), lambda i: (i, 0)),
    out_shape=jax.ShapeDtypeStruct((N_NODES, OUT_DIM), jnp.float32),
)


@jax.jit
def _run(x, edge_index, W1, b1, W2, b2):
    ei = edge_index.astype(jnp.int32)
    row3 = ei[0].reshape(NTILES, NCH, CH)
    col3 = ei[1].reshape(NTILES, NCH, CH)
    zeros2 = jnp.zeros((N_NODES, HID_DIM), jnp.float32)

    deg = _deg_call()(col3)
    hlin = _mm(x, W1)
    dinv, hp = _prescale(deg.reshape(N_NODES, 1), hlin)
    p1 = _prop_call()(hp, row3, col3, zeros2)
    h2lin, hp2 = _combine1(p1, hlin, dinv, b1.reshape(1, HID_DIM), W2)
    p2 = _prop_call()(hp2, row3, col3, zeros2)
    out = _combine2(p2, h2lin, dinv, b2.reshape(1, OUT_DIM))
    return out


def kernel(x, edge_index, W1, b1, W2, b2):
    return _run(x, edge_index, W1, b1, W2, b2)


# speedup vs baseline: 43.0045x; 1.6740x over previous
import functools

import jax
import jax.numpy as jnp
from jax import lax
from jax.experimental import pallas as pl
from jax.experimental.pallas import tpu as pltpu
from jax.experimental.pallas import tpu_sc as plsc

N_NODES = 10000
N_EDGES = 320000
IN_DIM = 128
HID_DIM = 64
OUT_DIM = 64

NC = 2
NS = 16
NTILES = NC * NS
EPT = N_EDGES // NTILES
CH = 100
NCH = EPT // CH
RPT = 624
RPT_LAST = N_NODES - 15 * RPT

@functools.cache
def _mesh():
    return plsc.VectorSubcoreMesh(core_axis_name="c", subcore_axis_name="s",
                                  num_cores=NC, num_subcores=NS)


def _deg_body(col3, deg_out, cidx, ones_v, zbuf, dbuf, shared_deg):
    c = lax.axis_index("c")
    s = lax.axis_index("s")
    for i in range(7):
        ones_v[pl.ds(i * 16, 16)] = jnp.full((16,), 1.0, jnp.float32)
    for i in range(63):
        zbuf[pl.ds(i * 16, 16)] = jnp.zeros((16,), jnp.float32)

    @pl.when(s < 10)
    def _():
        pltpu.sync_copy(zbuf.at[pl.ds(0, 1000)],
                        shared_deg.at[pl.ds(s * 1000, 1000)])

    plsc.subcore_barrier()

    def outer(k, carry):
        pltpu.sync_copy(col3.at[2 * s + k], cidx)

        def inner(j, carry2):
            pltpu.sync_copy(ones_v.at[pl.ds(0, CH)],
                            shared_deg.at[cidx.at[j]], add=True)
            return carry2

        lax.fori_loop(0, NCH, inner, 0)
        return carry

    lax.fori_loop(0, 2, outer, 0)
    plsc.subcore_barrier()

    @pl.when(s < 5)
    def _():
        base = c * 5000 + s * 1000
        pltpu.sync_copy(shared_deg.at[pl.ds(base, 1000)], dbuf)
        pltpu.sync_copy(dbuf, deg_out.at[pl.ds(base, 1000)])


@functools.cache
def _deg_call():
    return pl.kernel(
        _deg_body,
        out_type=jax.ShapeDtypeStruct((N_NODES,), jnp.float32),
        mesh=_mesh(),
        scratch_types=[
            pltpu.VMEM((NCH, CH), jnp.int32),
            pltpu.VMEM((112,), jnp.float32),
            pltpu.VMEM((1008,), jnp.float32),
            pltpu.VMEM((1000,), jnp.float32),
            pltpu.VMEM_SHARED((N_NODES,), jnp.float32),
        ],
        compiler_params=pltpu.CompilerParams(use_tc_tiling_on_sc=False),
    )


def _prop_body(hp, row3, col3, zeros2, out, ridx, cidx, rb0, rb1, rb2, rb3,
               gsem, shared_acc):
    c = lax.axis_index("c")
    s = lax.axis_index("s")
    wid = c * NS + s
    r0 = s * RPT
    bufs = (rb0, rb1, rb2, rb3)

    @pl.when(s < 15)
    def _():
        pltpu.sync_copy(zeros2.at[pl.ds(r0, RPT)],
                        shared_acc.at[pl.ds(r0, RPT)])

    @pl.when(s == 15)
    def _():
        pltpu.sync_copy(zeros2.at[pl.ds(15 * RPT, RPT_LAST)],
                        shared_acc.at[pl.ds(15 * RPT, RPT_LAST)])

    pltpu.sync_copy(row3.at[wid], ridx)
    pltpu.sync_copy(col3.at[wid], cidx)
    plsc.subcore_barrier()

    for b in range(4):
        pltpu.async_copy(hp.at[ridx.at[b]], bufs[b], gsem)

    def body(g, carry):
        for b in range(4):
            j = 4 * g + b
            pltpu.make_async_copy(hp.at[ridx.at[j]], bufs[b], gsem).wait()
            pltpu.sync_copy(bufs[b], shared_acc.at[cidx.at[j]], add=True)

            @pl.when(j + 4 < NCH)
            def _():
                pltpu.async_copy(hp.at[ridx.at[j + 4]], bufs[b], gsem)

        return carry

    lax.fori_loop(0, NCH // 4, body, 0)
    plsc.subcore_barrier()

    @pl.when(s < 15)
    def _():
        pltpu.sync_copy(shared_acc.at[pl.ds(r0, RPT)],
                        out.at[c, pl.ds(r0, RPT)])

    @pl.when(s == 15)
    def _():
        pltpu.sync_copy(shared_acc.at[pl.ds(15 * RPT, RPT_LAST)],
                        out.at[c, pl.ds(15 * RPT, RPT_LAST)])


@functools.cache
def _prop_call():
    return pl.kernel(
        _prop_body,
        out_type=jax.ShapeDtypeStruct((NC, N_NODES, HID_DIM), jnp.float32),
        mesh=_mesh(),
        scratch_types=[
            pltpu.VMEM((NCH, CH), jnp.int32),
            pltpu.VMEM((NCH, CH), jnp.int32),
            pltpu.VMEM((CH, HID_DIM), jnp.float32),
            pltpu.VMEM((CH, HID_DIM), jnp.float32),
            pltpu.VMEM((CH, HID_DIM), jnp.float32),
            pltpu.VMEM((CH, HID_DIM), jnp.float32),
            pltpu.SemaphoreType.DMA,
            pltpu.VMEM_SHARED((N_NODES, HID_DIM), jnp.float32),
        ],
        compiler_params=pltpu.CompilerParams(use_tc_tiling_on_sc=False),
    )


BLK = 2000
GRID = N_NODES // BLK


def _mm_body(x_ref, w_ref, o_ref):
    o_ref[...] = jnp.dot(x_ref[...], w_ref[...],
                         preferred_element_type=jnp.float32)


_mm = pl.pallas_call(
    _mm_body,
    grid=(GRID,),
    in_specs=[
        pl.BlockSpec((BLK, IN_DIM), lambda i: (i, 0)),
        pl.BlockSpec((IN_DIM, HID_DIM), lambda i: (0, 0)),
    ],
    out_specs=pl.BlockSpec((BLK, HID_DIM), lambda i: (i, 0)),
    out_shape=jax.ShapeDtypeStruct((N_NODES, HID_DIM), jnp.float32),
)


def _prescale_body(deg_ref, h_ref, dinv_ref, hp_ref):
    dinv = lax.rsqrt(deg_ref[...] + 1.0)
    dinv_ref[...] = dinv
    hp_ref[...] = h_ref[...] * dinv


_prescale = pl.pallas_call(
    _prescale_body,
    grid=(GRID,),
    in_specs=[
        pl.BlockSpec((BLK, 1), lambda i: (i, 0)),
        pl.BlockSpec((BLK, HID_DIM), lambda i: (i, 0)),
    ],
    out_specs=[
        pl.BlockSpec((BLK, 1), lambda i: (i, 0)),
        pl.BlockSpec((BLK, HID_DIM), lambda i: (i, 0)),
    ],
    out_shape=[
        jax.ShapeDtypeStruct((N_NODES, 1), jnp.float32),
        jax.ShapeDtypeStruct((N_NODES, HID_DIM), jnp.float32),
    ],
)


def _combine1_body(p_ref, h_ref, dinv_ref, b_ref, w_ref, h2_ref, hp2_ref):
    dinv = dinv_ref[...]
    psum = p_ref[0] + p_ref[1]
    out1 = jnp.maximum(dinv * psum + dinv * dinv * h_ref[...] + b_ref[...],
                       0.0)
    h2 = jnp.dot(out1, w_ref[...], preferred_element_type=jnp.float32)
    h2_ref[...] = h2
    hp2_ref[...] = h2 * dinv


_combine1 = pl.pallas_call(
    _combine1_body,
    grid=(GRID,),
    in_specs=[
        pl.BlockSpec((NC, BLK, HID_DIM), lambda i: (0, i, 0)),
        pl.BlockSpec((BLK, HID_DIM), lambda i: (i, 0)),
        pl.BlockSpec((BLK, 1), lambda i: (i, 0)),
        pl.BlockSpec((1, HID_DIM), lambda i: (0, 0)),
        pl.BlockSpec((HID_DIM, OUT_DIM), lambda i: (0, 0)),
    ],
    out_specs=[
        pl.BlockSpec((BLK, OUT_DIM), lambda i: (i, 0)),
        pl.BlockSpec((BLK, OUT_DIM), lambda i: (i, 0)),
    ],
    out_shape=[
        jax.ShapeDtypeStruct((N_NODES, OUT_DIM), jnp.float32),
        jax.ShapeDtypeStruct((N_NODES, OUT_DIM), jnp.float32),
    ],
)


def _combine2_body(p_ref, h_ref, dinv_ref, b_ref, o_ref):
    dinv = dinv_ref[...]
    psum = p_ref[0] + p_ref[1]
    o_ref[...] = dinv * psum + dinv * dinv * h_ref[...] + b_ref[...]


_combine2 = pl.pallas_call(
    _combine2_body,
    grid=(GRID,),
    in_specs=[
        pl.BlockSpec((NC, BLK, OUT_DIM), lambda i: (0, i, 0)),
        pl.BlockSpec((BLK, OUT_DIM), lambda i: (i, 0)),
        pl.BlockSpec((BLK, 1), lambda i: (i, 0)),
        pl.BlockSpec((1, OUT_DIM), lambda i: (0, 0)),
    ],
    out_specs=pl.BlockSpec((BLK, OUT_DIM), lambda i: (i, 0)),
    out_shape=jax.ShapeDtypeStruct((N_NODES, OUT_DIM), jnp.float32),
)


@jax.jit
def _run(x, edge_index, W1, b1, W2, b2):
    ei = edge_index.astype(jnp.int32)
    row3 = ei[0].reshape(NTILES, NCH, CH)
    col3 = ei[1].reshape(NTILES, NCH, CH)
    zeros2 = jnp.zeros((N_NODES, HID_DIM), jnp.float32)

    deg = _deg_call()(col3)
    hlin = _mm(x, W1)
    dinv, hp = _prescale(deg.reshape(N_NODES, 1), hlin)
    p1 = _prop_call()(hp, row3, col3, zeros2)
    h2lin, hp2 = _combine1(p1, hlin, dinv, b1.reshape(1, HID_DIM), W2)
    p2 = _prop_call()(hp2, row3, col3, zeros2)
    out = _combine2(p2, h2lin, dinv, b2.reshape(1, OUT_DIM))
    return out


def kernel(x, edge_index, W1, b1, W2, b2):
    return _run(x, edge_index, W1, b1, W2, b2)


# speedup vs baseline: 43.1837x; 1.0042x over previous
import functools

import jax
import jax.numpy as jnp
from jax import lax
from jax.experimental import pallas as pl
from jax.experimental.pallas import tpu as pltpu
from jax.experimental.pallas import tpu_sc as plsc

N_NODES = 10000
N_EDGES = 320000
IN_DIM = 128
HID_DIM = 64
OUT_DIM = 64

NC = 2
NS = 16
NTILES = NC * NS
EPT = N_EDGES // NTILES
CH = 125
NCH = EPT // CH
RPT = 624
RPT_LAST = N_NODES - 15 * RPT

@functools.cache
def _mesh():
    return plsc.VectorSubcoreMesh(core_axis_name="c", subcore_axis_name="s",
                                  num_cores=NC, num_subcores=NS)


def _deg_body(col3, deg_out, cidx, ones_v, zbuf, dbuf, shared_deg):
    c = lax.axis_index("c")
    s = lax.axis_index("s")
    for i in range(8):
        ones_v[pl.ds(i * 16, 16)] = jnp.full((16,), 1.0, jnp.float32)
    for i in range(63):
        zbuf[pl.ds(i * 16, 16)] = jnp.zeros((16,), jnp.float32)

    @pl.when(s < 10)
    def _():
        pltpu.sync_copy(zbuf.at[pl.ds(0, 1000)],
                        shared_deg.at[pl.ds(s * 1000, 1000)])

    plsc.subcore_barrier()

    def outer(k, carry):
        pltpu.sync_copy(col3.at[2 * s + k], cidx)

        def inner(j, carry2):
            pltpu.sync_copy(ones_v.at[pl.ds(0, CH)],
                            shared_deg.at[cidx.at[j]], add=True)
            return carry2

        lax.fori_loop(0, NCH, inner, 0)
        return carry

    lax.fori_loop(0, 2, outer, 0)
    plsc.subcore_barrier()

    @pl.when(s < 5)
    def _():
        base = c * 5000 + s * 1000
        pltpu.sync_copy(shared_deg.at[pl.ds(base, 1000)], dbuf)
        pltpu.sync_copy(dbuf, deg_out.at[pl.ds(base, 1000)])


@functools.cache
def _deg_call():
    return pl.kernel(
        _deg_body,
        out_type=jax.ShapeDtypeStruct((N_NODES,), jnp.float32),
        mesh=_mesh(),
        scratch_types=[
            pltpu.VMEM((NCH, CH), jnp.int32),
            pltpu.VMEM((128,), jnp.float32),
            pltpu.VMEM((1008,), jnp.float32),
            pltpu.VMEM((1000,), jnp.float32),
            pltpu.VMEM_SHARED((N_NODES,), jnp.float32),
        ],
        compiler_params=pltpu.CompilerParams(use_tc_tiling_on_sc=False),
    )


def _prop_body(hp, row3, col3, zeros2, out, ridx, cidx, rb0, rb1, rb2, rb3,
               gsem, shared_acc):
    c = lax.axis_index("c")
    s = lax.axis_index("s")
    wid = c * NS + s
    r0 = s * RPT
    bufs = (rb0, rb1, rb2, rb3)

    @pl.when(s < 15)
    def _():
        pltpu.sync_copy(zeros2.at[pl.ds(r0, RPT)],
                        shared_acc.at[pl.ds(r0, RPT)])

    @pl.when(s == 15)
    def _():
        pltpu.sync_copy(zeros2.at[pl.ds(15 * RPT, RPT_LAST)],
                        shared_acc.at[pl.ds(15 * RPT, RPT_LAST)])

    pltpu.sync_copy(row3.at[wid], ridx)
    pltpu.sync_copy(col3.at[wid], cidx)
    plsc.subcore_barrier()

    for b in range(4):
        pltpu.async_copy(hp.at[ridx.at[b]], bufs[b], gsem)

    def body(g, carry):
        for b in range(4):
            j = 4 * g + b
            pltpu.make_async_copy(hp.at[ridx.at[j]], bufs[b], gsem).wait()
            pltpu.sync_copy(bufs[b], shared_acc.at[cidx.at[j]], add=True)

            @pl.when(j + 4 < NCH)
            def _():
                pltpu.async_copy(hp.at[ridx.at[j + 4]], bufs[b], gsem)

        return carry

    lax.fori_loop(0, NCH // 4, body, 0)
    plsc.subcore_barrier()

    @pl.when(s < 15)
    def _():
        pltpu.sync_copy(shared_acc.at[pl.ds(r0, RPT)],
                        out.at[c, pl.ds(r0, RPT)])

    @pl.when(s == 15)
    def _():
        pltpu.sync_copy(shared_acc.at[pl.ds(15 * RPT, RPT_LAST)],
                        out.at[c, pl.ds(15 * RPT, RPT_LAST)])


@functools.cache
def _prop_call():
    return pl.kernel(
        _prop_body,
        out_type=jax.ShapeDtypeStruct((NC, N_NODES, HID_DIM), jnp.float32),
        mesh=_mesh(),
        scratch_types=[
            pltpu.VMEM((NCH, CH), jnp.int32),
            pltpu.VMEM((NCH, CH), jnp.int32),
            pltpu.VMEM((CH, HID_DIM), jnp.float32),
            pltpu.VMEM((CH, HID_DIM), jnp.float32),
            pltpu.VMEM((CH, HID_DIM), jnp.float32),
            pltpu.VMEM((CH, HID_DIM), jnp.float32),
            pltpu.SemaphoreType.DMA,
            pltpu.VMEM_SHARED((N_NODES, HID_DIM), jnp.float32),
        ],
        compiler_params=pltpu.CompilerParams(use_tc_tiling_on_sc=False),
    )


BLK = 2000
GRID = N_NODES // BLK


def _mmps_body(x_ref, w_ref, deg_ref, hlin_ref, dinv_ref, hp_ref):
    hlin = jnp.dot(x_ref[...], w_ref[...], preferred_element_type=jnp.float32)
    dinv = lax.rsqrt(deg_ref[...] + 1.0)
    hlin_ref[...] = hlin
    dinv_ref[...] = dinv
    hp_ref[...] = hlin * dinv


_mmps = pl.pallas_call(
    _mmps_body,
    grid=(GRID,),
    in_specs=[
        pl.BlockSpec((BLK, IN_DIM), lambda i: (i, 0)),
        pl.BlockSpec((IN_DIM, HID_DIM), lambda i: (0, 0)),
        pl.BlockSpec((BLK, 1), lambda i: (i, 0)),
    ],
    out_specs=[
        pl.BlockSpec((BLK, HID_DIM), lambda i: (i, 0)),
        pl.BlockSpec((BLK, 1), lambda i: (i, 0)),
        pl.BlockSpec((BLK, HID_DIM), lambda i: (i, 0)),
    ],
    out_shape=[
        jax.ShapeDtypeStruct((N_NODES, HID_DIM), jnp.float32),
        jax.ShapeDtypeStruct((N_NODES, 1), jnp.float32),
        jax.ShapeDtypeStruct((N_NODES, HID_DIM), jnp.float32),
    ],
)


def _combine1_body(p_ref, h_ref, dinv_ref, b_ref, w_ref, h2_ref, hp2_ref):
    dinv = dinv_ref[...]
    psum = p_ref[0] + p_ref[1]
    out1 = jnp.maximum(dinv * psum + dinv * dinv * h_ref[...] + b_ref[...],
                       0.0)
    h2 = jnp.dot(out1, w_ref[...], preferred_element_type=jnp.float32)
    h2_ref[...] = h2
    hp2_ref[...] = h2 * dinv


_combine1 = pl.pallas_call(
    _combine1_body,
    grid=(GRID,),
    in_specs=[
        pl.BlockSpec((NC, BLK, HID_DIM), lambda i: (0, i, 0)),
        pl.BlockSpec((BLK, HID_DIM), lambda i: (i, 0)),
        pl.BlockSpec((BLK, 1), lambda i: (i, 0)),
        pl.BlockSpec((1, HID_DIM), lambda i: (0, 0)),
        pl.BlockSpec((HID_DIM, OUT_DIM), lambda i: (0, 0)),
    ],
    out_specs=[
        pl.BlockSpec((BLK, OUT_DIM), lambda i: (i, 0)),
        pl.BlockSpec((BLK, OUT_DIM), lambda i: (i, 0)),
    ],
    out_shape=[
        jax.ShapeDtypeStruct((N_NODES, OUT_DIM), jnp.float32),
        jax.ShapeDtypeStruct((N_NODES, OUT_DIM), jnp.float32),
    ],
)


def _combine2_body(p_ref, h_ref, dinv_ref, b_ref, o_ref):
    dinv = dinv_ref[...]
    psum = p_ref[0] + p_ref[1]
    o_ref[...] = dinv * psum + dinv * dinv * h_ref[...] + b_ref[...]


_combine2 = pl.pallas_call(
    _combine2_body,
    grid=(GRID,),
    in_specs=[
        pl.BlockSpec((NC, BLK, OUT_DIM), lambda i: (0, i, 0)),
        pl.BlockSpec((BLK, OUT_DIM), lambda i: (i, 0)),
        pl.BlockSpec((BLK, 1), lambda i: (i, 0)),
        pl.BlockSpec((1, OUT_DIM), lambda i: (0, 0)),
    ],
    out_specs=pl.BlockSpec((BLK, OUT_DIM), lambda i: (i, 0)),
    out_shape=jax.ShapeDtypeStruct((N_NODES, OUT_DIM), jnp.float32),
)


@jax.jit
def _run(x, edge_index, W1, b1, W2, b2):
    ei = edge_index.astype(jnp.int32)
    row3 = ei[0].reshape(NTILES, NCH, CH)
    col3 = ei[1].reshape(NTILES, NCH, CH)
    zeros2 = jnp.zeros((N_NODES, HID_DIM), jnp.float32)

    deg = _deg_call()(col3)
    hlin, dinv, hp = _mmps(x, W1, deg.reshape(N_NODES, 1))
    p1 = _prop_call()(hp, row3, col3, zeros2)
    h2lin, hp2 = _combine1(p1, hlin, dinv, b1.reshape(1, HID_DIM), W2)
    p2 = _prop_call()(hp2, row3, col3, zeros2)
    out = _combine2(p2, h2lin, dinv, b2.reshape(1, OUT_DIM))
    return out


def kernel(x, edge_index, W1, b1, W2, b2):
    return _run(x, edge_index, W1, b1, W2, b2)


# speedup vs baseline: 45.3505x; 1.0502x over previous
import functools

import jax
import jax.numpy as jnp
from jax import lax
from jax.experimental import pallas as pl
from jax.experimental.pallas import tpu as pltpu
from jax.experimental.pallas import tpu_sc as plsc

N_NODES = 10000
N_EDGES = 320000
IN_DIM = 128
HID_DIM = 64
OUT_DIM = 64

NC = 2
NS = 16
NTILES = NC * NS
EPT = N_EDGES // NTILES
CH = 125
NCH = EPT // CH
RPT = 624
RPT_LAST = N_NODES - 15 * RPT

@functools.cache
def _mesh():
    return plsc.VectorSubcoreMesh(core_axis_name="c", subcore_axis_name="s",
                                  num_cores=NC, num_subcores=NS)


_RING = 8


def _deg_body(col_deg, deg_out, cidx, ones_v, zbuf, dbuf, ssem, shared_deg):
    c = lax.axis_index("c")
    s = lax.axis_index("s")
    for i in range(8):
        ones_v[pl.ds(i * 16, 16)] = jnp.full((16,), 1.0, jnp.float32)
    for i in range(63):
        zbuf[pl.ds(i * 16, 16)] = jnp.zeros((16,), jnp.float32)

    @pl.when(s < 10)
    def _():
        pltpu.sync_copy(zbuf.at[pl.ds(0, 1000)],
                        shared_deg.at[pl.ds(s * 1000, 1000)])

    pltpu.sync_copy(col_deg.at[s], cidx)
    plsc.subcore_barrier()

    def body(j, carry):
        @pl.when(j >= _RING)
        def _():
            pltpu.make_async_copy(ones_v.at[pl.ds(0, CH)],
                                  shared_deg.at[cidx.at[0]], ssem).wait()

        pltpu.async_copy(ones_v.at[pl.ds(0, CH)],
                         shared_deg.at[cidx.at[j]], ssem, add=True)
        return carry

    lax.fori_loop(0, 2 * NCH, body, 0)
    for _ in range(_RING):
        pltpu.make_async_copy(ones_v.at[pl.ds(0, CH)],
                              shared_deg.at[cidx.at[0]], ssem).wait()
    plsc.subcore_barrier()

    @pl.when(s < 5)
    def _():
        base = c * 5000 + s * 1000
        pltpu.sync_copy(shared_deg.at[pl.ds(base, 1000)], dbuf)
        pltpu.sync_copy(dbuf, deg_out.at[pl.ds(base, 1000)])


@functools.cache
def _deg_call():
    return pl.kernel(
        _deg_body,
        out_type=jax.ShapeDtypeStruct((N_NODES,), jnp.float32),
        mesh=_mesh(),
        scratch_types=[
            pltpu.VMEM((2 * NCH, CH), jnp.int32),
            pltpu.VMEM((128,), jnp.float32),
            pltpu.VMEM((1008,), jnp.float32),
            pltpu.VMEM((1000,), jnp.float32),
            pltpu.SemaphoreType.DMA,
            pltpu.VMEM_SHARED((N_NODES,), jnp.float32),
        ],
        compiler_params=pltpu.CompilerParams(use_tc_tiling_on_sc=False),
    )


def _prop_body(hp, row3, col3, zeros2, out, ridx, cidx, rb0, rb1, rb2, rb3,
               gsem, shared_acc):
    c = lax.axis_index("c")
    s = lax.axis_index("s")
    wid = c * NS + s
    r0 = s * RPT
    bufs = (rb0, rb1, rb2, rb3)

    @pl.when(s < 15)
    def _():
        pltpu.sync_copy(zeros2.at[pl.ds(r0, RPT)],
                        shared_acc.at[pl.ds(r0, RPT)])

    @pl.when(s == 15)
    def _():
        pltpu.sync_copy(zeros2.at[pl.ds(15 * RPT, RPT_LAST)],
                        shared_acc.at[pl.ds(15 * RPT, RPT_LAST)])

    pltpu.sync_copy(row3.at[wid], ridx)
    pltpu.sync_copy(col3.at[wid], cidx)
    plsc.subcore_barrier()

    for b in range(4):
        pltpu.async_copy(hp.at[ridx.at[b]], bufs[b], gsem)

    def body(g, carry):
        for b in range(4):
            j = 4 * g + b
            pltpu.make_async_copy(hp.at[ridx.at[j]], bufs[b], gsem).wait()
            pltpu.sync_copy(bufs[b], shared_acc.at[cidx.at[j]], add=True)

            @pl.when(j + 4 < NCH)
            def _():
                pltpu.async_copy(hp.at[ridx.at[j + 4]], bufs[b], gsem)

        return carry

    lax.fori_loop(0, NCH // 4, body, 0)
    plsc.subcore_barrier()

    @pl.when(s < 15)
    def _():
        pltpu.sync_copy(shared_acc.at[pl.ds(r0, RPT)],
                        out.at[c, pl.ds(r0, RPT)])

    @pl.when(s == 15)
    def _():
        pltpu.sync_copy(shared_acc.at[pl.ds(15 * RPT, RPT_LAST)],
                        out.at[c, pl.ds(15 * RPT, RPT_LAST)])


@functools.cache
def _prop_call():
    return pl.kernel(
        _prop_body,
        out_type=jax.ShapeDtypeStruct((NC, N_NODES, HID_DIM), jnp.float32),
        mesh=_mesh(),
        scratch_types=[
            pltpu.VMEM((NCH, CH), jnp.int32),
            pltpu.VMEM((NCH, CH), jnp.int32),
            pltpu.VMEM((CH, HID_DIM), jnp.float32),
            pltpu.VMEM((CH, HID_DIM), jnp.float32),
            pltpu.VMEM((CH, HID_DIM), jnp.float32),
            pltpu.VMEM((CH, HID_DIM), jnp.float32),
            pltpu.SemaphoreType.DMA,
            pltpu.VMEM_SHARED((N_NODES, HID_DIM), jnp.float32),
        ],
        compiler_params=pltpu.CompilerParams(use_tc_tiling_on_sc=False),
    )


BLK = 2000
GRID = N_NODES // BLK


def _mmps_body(x_ref, w_ref, deg_ref, hlin_ref, dinv_ref, hp_ref):
    hlin = jnp.dot(x_ref[...], w_ref[...], preferred_element_type=jnp.float32)
    dinv = lax.rsqrt(deg_ref[...] + 1.0)
    hlin_ref[...] = hlin
    dinv_ref[...] = dinv
    hp_ref[...] = hlin * dinv


_mmps = pl.pallas_call(
    _mmps_body,
    grid=(GRID,),
    in_specs=[
        pl.BlockSpec((BLK, IN_DIM), lambda i: (i, 0)),
        pl.BlockSpec((IN_DIM, HID_DIM), lambda i: (0, 0)),
        pl.BlockSpec((BLK, 1), lambda i: (i, 0)),
    ],
    out_specs=[
        pl.BlockSpec((BLK, HID_DIM), lambda i: (i, 0)),
        pl.BlockSpec((BLK, 1), lambda i: (i, 0)),
        pl.BlockSpec((BLK, HID_DIM), lambda i: (i, 0)),
    ],
    out_shape=[
        jax.ShapeDtypeStruct((N_NODES, HID_DIM), jnp.float32),
        jax.ShapeDtypeStruct((N_NODES, 1), jnp.float32),
        jax.ShapeDtypeStruct((N_NODES, HID_DIM), jnp.float32),
    ],
)


def _combine1_body(p_ref, h_ref, dinv_ref, b_ref, w_ref, h2_ref, hp2_ref):
    dinv = dinv_ref[...]
    psum = p_ref[0] + p_ref[1]
    out1 = jnp.maximum(dinv * psum + dinv * dinv * h_ref[...] + b_ref[...],
                       0.0)
    h2 = jnp.dot(out1, w_ref[...], preferred_element_type=jnp.float32)
    h2_ref[...] = h2
    hp2_ref[...] = h2 * dinv


_combine1 = pl.pallas_call(
    _combine1_body,
    grid=(GRID,),
    in_specs=[
        pl.BlockSpec((NC, BLK, HID_DIM), lambda i: (0, i, 0)),
        pl.BlockSpec((BLK, HID_DIM), lambda i: (i, 0)),
        pl.BlockSpec((BLK, 1), lambda i: (i, 0)),
        pl.BlockSpec((1, HID_DIM), lambda i: (0, 0)),
        pl.BlockSpec((HID_DIM, OUT_DIM), lambda i: (0, 0)),
    ],
    out_specs=[
        pl.BlockSpec((BLK, OUT_DIM), lambda i: (i, 0)),
        pl.BlockSpec((BLK, OUT_DIM), lambda i: (i, 0)),
    ],
    out_shape=[
        jax.ShapeDtypeStruct((N_NODES, OUT_DIM), jnp.float32),
        jax.ShapeDtypeStruct((N_NODES, OUT_DIM), jnp.float32),
    ],
)


def _combine2_body(p_ref, h_ref, dinv_ref, b_ref, o_ref):
    dinv = dinv_ref[...]
    psum = p_ref[0] + p_ref[1]
    o_ref[...] = dinv * psum + dinv * dinv * h_ref[...] + b_ref[...]


_combine2 = pl.pallas_call(
    _combine2_body,
    grid=(GRID,),
    in_specs=[
        pl.BlockSpec((NC, BLK, OUT_DIM), lambda i: (0, i, 0)),
        pl.BlockSpec((BLK, OUT_DIM), lambda i: (i, 0)),
        pl.BlockSpec((BLK, 1), lambda i: (i, 0)),
        pl.BlockSpec((1, OUT_DIM), lambda i: (0, 0)),
    ],
    out_specs=pl.BlockSpec((BLK, OUT_DIM), lambda i: (i, 0)),
    out_shape=jax.ShapeDtypeStruct((N_NODES, OUT_DIM), jnp.float32),
)


@jax.jit
def _run(x, edge_index, W1, b1, W2, b2):
    ei = edge_index.astype(jnp.int32)
    row3 = ei[0].reshape(NTILES, NCH, CH)
    col3 = ei[1].reshape(NTILES, NCH, CH)
    col_deg = ei[1].reshape(NS, 2 * NCH, CH)
    zeros2 = jnp.zeros((N_NODES, HID_DIM), jnp.float32)

    deg = _deg_call()(col_deg)
    hlin, dinv, hp = _mmps(x, W1, deg.reshape(N_NODES, 1))
    p1 = _prop_call()(hp, row3, col3, zeros2)
    h2lin, hp2 = _combine1(p1, hlin, dinv, b1.reshape(1, HID_DIM), W2)
    p2 = _prop_call()(hp2, row3, col3, zeros2)
    out = _combine2(p2, h2lin, dinv, b2.reshape(1, OUT_DIM))
    return out


def kernel(x, edge_index, W1, b1, W2, b2):
    return _run(x, edge_index, W1, b1, W2, b2)
